# baseline (speedup 1.0000x reference)
import functools

import jax
import jax.numpy as jnp
from jax import lax
from jax.experimental import pallas as pl
from jax.experimental.pallas import tpu as pltpu
from jax.experimental.pallas import tpu_sc as plsc

B = 1024
T_LEN = 200
EMBED = 128
NC = 2
NS = 16
NW = NC * NS
BT = B * T_LEN
RPW = BT // NW
CHUNK = 128
NCH = RPW // CHUNK
NVR = EMBED // 16
NBUF = 5


def _sc_body(idx_hbm, posidx_hbm, tok_hbm, pos_hbm, out_hbm,
             idx_v, posidx_v, pos_v, bufs, sgs, sos):
    wid = lax.axis_index("s") * NC + lax.axis_index("c")
    base = wid * RPW

    pltpu.sync_copy(idx_hbm.at[pl.ds(base, RPW)], idx_v)
    pltpu.sync_copy(posidx_hbm, posidx_v)
    p1 = pltpu.async_copy(pos_hbm.at[posidx_v.at[pl.ds(0, 128)]],
                          pos_v.at[pl.ds(0, 128)], sgs[0])
    p2 = pltpu.async_copy(pos_hbm.at[posidx_v.at[pl.ds(128, T_LEN - 128)]],
                          pos_v.at[pl.ds(128, T_LEN - 128)], sgs[0])
    p1.wait()
    p2.wait()

    def start_gather(ci, buf, sem):
        return pltpu.async_copy(tok_hbm.at[idx_v.at[pl.ds(ci * CHUNK, CHUNK)]],
                                buf, sem)

    def add_run(buf, r0, p0, n):
        def body(i, _):
            for j in range(NVR):
                sl = pl.ds(j * 16, 16)
                buf[r0 + i, sl] = buf[r0 + i, sl] + pos_v[p0 + i, sl]
            return 0
        lax.fori_loop(0, n, body, 0)

    pend_g = start_gather(0, bufs[0], sgs[0])
    pend_o = [None] * NBUF
    for ci in range(NCH):
        pb = ci % NBUF
        if ci + 1 < NCH:
            nb = (ci + 1) % NBUF
            if pend_o[nb] is not None:
                pend_o[nb].wait()
                pend_o[nb] = None
            next_g = start_gather(ci + 1, bufs[nb], sgs[nb])
        pend_g.wait()
        buf = bufs[pb]
        phase = (ci * CHUNK) % T_LEN
        run1 = min(CHUNK, T_LEN - phase)
        add_run(buf, 0, phase, run1)
        if run1 < CHUNK:
            add_run(buf, run1, 0, CHUNK - run1)
        pend_o[pb] = pltpu.async_copy(
            buf, out_hbm.at[pl.ds(base + ci * CHUNK, CHUNK)], sos[pb])
        if ci + 1 < NCH:
            pend_g = next_g
    for po in pend_o:
        if po is not None:
            po.wait()


@jax.jit
def _combined_input_sc(idx_flat, pos_idx, token_table, pos_table):
    mesh = plsc.VectorSubcoreMesh(core_axis_name="c", subcore_axis_name="s")
    call = pl.kernel(
        _sc_body,
        out_type=jax.ShapeDtypeStruct((BT, EMBED), jnp.float32),
        mesh=mesh,
        scratch_types=[
            pltpu.VMEM((RPW,), jnp.int32),
            pltpu.VMEM((T_LEN,), jnp.int32),
            pltpu.VMEM((T_LEN, EMBED), jnp.float32),
            [pltpu.VMEM((CHUNK, EMBED), jnp.float32) for _ in range(NBUF)],
            [pltpu.SemaphoreType.DMA for _ in range(NBUF)],
            [pltpu.SemaphoreType.DMA for _ in range(NBUF)],
        ],
    )
    return call(idx_flat, pos_idx, token_table, pos_table)


def kernel(idx, T, token_table, pos_table):
    idx_flat = idx.reshape(BT).astype(jnp.int32)
    pos_idx = jnp.minimum(jnp.arange(T_LEN, dtype=jnp.int32),
                          jnp.asarray(T, jnp.int32) - 1)
    out = _combined_input_sc(idx_flat, pos_idx, token_table, pos_table)
    return out.reshape(B, T_LEN, EMBED)


# speedup vs baseline: 1.0685x; 1.0685x over previous
import functools

import jax
import jax.numpy as jnp
from jax import lax
from jax.experimental import pallas as pl
from jax.experimental.pallas import tpu as pltpu
from jax.experimental.pallas import tpu_sc as plsc

B = 1024
T_LEN = 200
EMBED = 128
NC = 2
NS = 16
NW = NC * NS
BT = B * T_LEN
RPW = BT // NW
CHUNK = 200
NCH = RPW // CHUNK
NVR = EMBED // 16
NBUF = 3


def _sc_body(idx_hbm, posidx_hbm, tok_hbm, pos_hbm, out_hbm,
             idx_v, posidx_v, pos_v, bufs, sgs, sos):
    wid = lax.axis_index("s") * NC + lax.axis_index("c")
    base = wid * RPW

    pltpu.sync_copy(idx_hbm.at[pl.ds(base, RPW)], idx_v)
    pltpu.sync_copy(posidx_hbm, posidx_v)
    p1 = pltpu.async_copy(pos_hbm.at[posidx_v.at[pl.ds(0, 128)]],
                          pos_v.at[pl.ds(0, 128)], sgs[0])
    p2 = pltpu.async_copy(pos_hbm.at[posidx_v.at[pl.ds(128, T_LEN - 128)]],
                          pos_v.at[pl.ds(128, T_LEN - 128)], sgs[0])
    p1.wait()
    p2.wait()

    def start_gather(ci, buf, sem):
        off = ci * CHUNK
        a = pltpu.async_copy(tok_hbm.at[idx_v.at[pl.ds(off, 128)]],
                             buf.at[pl.ds(0, 128)], sem)
        b = pltpu.async_copy(tok_hbm.at[idx_v.at[pl.ds(off + 128, CHUNK - 128)]],
                             buf.at[pl.ds(128, CHUNK - 128)], sem)
        return a, b

    def add_run(buf, r0, p0, n):
        def body(i, _):
            for j in range(NVR):
                sl = pl.ds(j * 16, 16)
                buf[r0 + i, sl] = buf[r0 + i, sl] + pos_v[p0 + i, sl]
            return 0
        lax.fori_loop(0, n, body, 0)

    pend_g = start_gather(0, bufs[0], sgs[0])
    pend_o = [None] * NBUF
    for ci in range(NCH):
        pb = ci % NBUF
        if ci + 1 < NCH:
            nb = (ci + 1) % NBUF
            if pend_o[nb] is not None:
                pend_o[nb].wait()
                pend_o[nb] = None
            next_g = start_gather(ci + 1, bufs[nb], sgs[nb])
        for h in pend_g:
            h.wait()
        buf = bufs[pb]
        phase = (ci * CHUNK) % T_LEN
        run1 = min(CHUNK, T_LEN - phase)
        add_run(buf, 0, phase, run1)
        if run1 < CHUNK:
            add_run(buf, run1, 0, CHUNK - run1)
        pend_o[pb] = pltpu.async_copy(
            buf, out_hbm.at[pl.ds(base + ci * CHUNK, CHUNK)], sos[pb])
        if ci + 1 < NCH:
            pend_g = next_g
    for po in pend_o:
        if po is not None:
            po.wait()


@jax.jit
def _combined_input_sc(idx_flat, pos_idx, token_table, pos_table):
    mesh = plsc.VectorSubcoreMesh(core_axis_name="c", subcore_axis_name="s")
    call = pl.kernel(
        _sc_body,
        out_type=jax.ShapeDtypeStruct((BT, EMBED), jnp.float32),
        mesh=mesh,
        scratch_types=[
            pltpu.VMEM((RPW,), jnp.int32),
            pltpu.VMEM((T_LEN,), jnp.int32),
            pltpu.VMEM((T_LEN, EMBED), jnp.float32),
            [pltpu.VMEM((CHUNK, EMBED), jnp.float32) for _ in range(NBUF)],
            [pltpu.SemaphoreType.DMA for _ in range(NBUF)],
            [pltpu.SemaphoreType.DMA for _ in range(NBUF)],
        ],
    )
    return call(idx_flat, pos_idx, token_table, pos_table)


def kernel(idx, T, token_table, pos_table):
    idx_flat = idx.reshape(BT).astype(jnp.int32)
    pos_idx = jnp.minimum(jnp.arange(T_LEN, dtype=jnp.int32),
                          jnp.asarray(T, jnp.int32) - 1)
    out = _combined_input_sc(idx_flat, pos_idx, token_table, pos_table)
    return out.reshape(B, T_LEN, EMBED)


# speedup vs baseline: 1.0774x; 1.0083x over previous
import functools

import jax
import jax.numpy as jnp
from jax import lax
from jax.experimental import pallas as pl
from jax.experimental.pallas import tpu as pltpu
from jax.experimental.pallas import tpu_sc as plsc

B = 1024
T_LEN = 200
EMBED = 128
NC = 2
NS = 16
NW = NC * NS
BT = B * T_LEN
RPW = BT // NW
CHUNK = 200
NCH = RPW // CHUNK
NVR = EMBED // 16
NBUF = 3


def _sc_body(idx_hbm, t_hbm, tok_hbm, pos_hbm, out_hbm,
             idx_v, t_v, posidx_v, pos_v, bufs, sgs, sos):
    wid = lax.axis_index("s") * NC + lax.axis_index("c")
    base = wid * RPW

    idx_cp = pltpu.async_copy(idx_hbm.at[pl.ds(base, RPW)], idx_v, sos[0])
    pltpu.sync_copy(t_hbm, t_v.at[pl.ds(0, 1)])
    dnums = lax.GatherDimensionNumbers(
        offset_dims=(), collapsed_slice_dims=(0,), start_index_map=(0,))
    t16 = lax.gather(t_v[...], jnp.zeros((16, 1), jnp.int32), dnums,
                     slice_sizes=(1,),
                     mode=lax.GatherScatterMode.PROMISE_IN_BOUNDS)
    for k in range(T_LEN // 16 + 1):
        v = jnp.minimum(lax.iota(jnp.int32, 16) + 16 * k, t16 - 1)
        posidx_v[pl.ds(16 * k, 16)] = v
    p1 = pltpu.async_copy(pos_hbm.at[posidx_v.at[pl.ds(0, 128)]],
                          pos_v.at[pl.ds(0, 128)], sgs[0])
    p2 = pltpu.async_copy(pos_hbm.at[posidx_v.at[pl.ds(128, T_LEN - 128)]],
                          pos_v.at[pl.ds(128, T_LEN - 128)], sgs[0])
    p1.wait()
    p2.wait()
    idx_cp.wait()

    def start_gather(ci, buf, sem):
        off = ci * CHUNK
        a = pltpu.async_copy(tok_hbm.at[idx_v.at[pl.ds(off, 128)]],
                             buf.at[pl.ds(0, 128)], sem)
        b = pltpu.async_copy(tok_hbm.at[idx_v.at[pl.ds(off + 128, CHUNK - 128)]],
                             buf.at[pl.ds(128, CHUNK - 128)], sem)
        return a, b

    def add_run(buf, r0, p0, n):
        def body(i, _):
            for j in range(NVR):
                sl = pl.ds(j * 16, 16)
                buf[r0 + i, sl] = buf[r0 + i, sl] + pos_v[p0 + i, sl]
            return 0
        lax.fori_loop(0, n, body, 0)

    pend_g = start_gather(0, bufs[0], sgs[0])
    pend_o = [None] * NBUF
    for ci in range(NCH):
        pb = ci % NBUF
        if ci + 1 < NCH:
            nb = (ci + 1) % NBUF
            if pend_o[nb] is not None:
                pend_o[nb].wait()
                pend_o[nb] = None
            next_g = start_gather(ci + 1, bufs[nb], sgs[nb])
        for h in pend_g:
            h.wait()
        buf = bufs[pb]
        phase = (ci * CHUNK) % T_LEN
        run1 = min(CHUNK, T_LEN - phase)
        add_run(buf, 0, phase, run1)
        if run1 < CHUNK:
            add_run(buf, run1, 0, CHUNK - run1)
        pend_o[pb] = pltpu.async_copy(
            buf, out_hbm.at[pl.ds(base + ci * CHUNK, CHUNK)], sos[pb])
        if ci + 1 < NCH:
            pend_g = next_g
    for po in pend_o:
        if po is not None:
            po.wait()


@jax.jit
def _combined_input_sc(idx_flat, t_arr, token_table, pos_table):
    mesh = plsc.VectorSubcoreMesh(core_axis_name="c", subcore_axis_name="s")
    call = pl.kernel(
        _sc_body,
        out_type=jax.ShapeDtypeStruct((BT, EMBED), jnp.float32),
        mesh=mesh,
        scratch_types=[
            pltpu.VMEM((RPW,), jnp.int32),
            pltpu.VMEM((16,), jnp.int32),
            pltpu.VMEM((T_LEN + 16, ), jnp.int32),
            pltpu.VMEM((T_LEN, EMBED), jnp.float32),
            [pltpu.VMEM((CHUNK, EMBED), jnp.float32) for _ in range(NBUF)],
            [pltpu.SemaphoreType.DMA for _ in range(NBUF)],
            [pltpu.SemaphoreType.DMA for _ in range(NBUF)],
        ],
    )
    return call(idx_flat, t_arr, token_table, pos_table)


def kernel(idx, T, token_table, pos_table):
    idx_flat = idx.reshape(BT).astype(jnp.int32)
    t_arr = jnp.asarray(T, jnp.int32).reshape(1)
    out = _combined_input_sc(idx_flat, t_arr, token_table, pos_table)
    return out.reshape(B, T_LEN, EMBED)
